# Initial kernel scaffold; baseline (speedup 1.0000x reference)
import jax
import jax.numpy as jnp
from jax.experimental import pallas as pl


def kernel(x, lut_weights):
    raise NotImplementedError("write your pallas kernel here")


# speedup vs baseline: 390.9645x; 390.9645x over previous
import functools

import jax
import jax.numpy as jnp
from jax import lax
from jax.experimental import pallas as pl
from jax.experimental.pallas import tpu as pltpu
from jax.experimental.pallas import tpu_sc as plsc

BATCH = 16384
NUM_INPUTS = 2048
TUPLE_SIZE = 8
NUM_LUTS = NUM_INPUTS // TUPLE_SIZE
LUT_ENTRIES = 1 << TUPLE_SIZE


TC_ROWS = 512


def _pack_body(x_ref, p_ref, out_ref):
    xb = x_ref[...].astype(jnp.bfloat16)
    addr = jnp.dot(xb, p_ref[...], preferred_element_type=jnp.float32)
    toff = lax.broadcasted_iota(jnp.int32, (TC_ROWS, NUM_LUTS), 1) * LUT_ENTRIES
    out_ref[...] = addr.astype(jnp.int32) + toff


def _pack_addresses(x, p):
    return pl.pallas_call(
        _pack_body,
        grid=(BATCH // TC_ROWS,),
        in_specs=[
            pl.BlockSpec((TC_ROWS, NUM_INPUTS), lambda i: (i, 0)),
            pl.BlockSpec((NUM_INPUTS, NUM_LUTS), lambda i: (0, 0)),
        ],
        out_specs=pl.BlockSpec((TC_ROWS, NUM_LUTS), lambda i: (i, 0)),
        out_shape=jax.ShapeDtypeStruct((BATCH, NUM_LUTS), jnp.int32),
    )(x, p)


NW = 32
ROWS_PER_TILE = BATCH // NW
CHUNK = 32
LANES = 16
TABLE = NUM_LUTS * LUT_ENTRIES


def _sc_body(fidx_hbm, lut_hbm, out_hbm, lut_v, idx_v, out_v):
    wid = lax.axis_index("s") * 2 + lax.axis_index("c")
    base_row = wid * ROWS_PER_TILE
    pltpu.sync_copy(lut_hbm, lut_v)

    def chunk_body(ci, _):
        row0 = base_row + ci * CHUNK
        pltpu.sync_copy(
            fidx_hbm.at[pl.ds(row0 * NUM_LUTS, CHUNK * NUM_LUTS)], idx_v
        )

        def group_body(g, _):
            vec = jnp.zeros((LANES,), jnp.float32)
            for r in range(LANES):
                rbase = (g * LANES + r) * NUM_LUTS
                acc = jnp.zeros((LANES,), jnp.float32)
                for j in range(NUM_LUTS // LANES):
                    idx = idx_v[pl.ds(rbase + j * LANES, LANES)]
                    acc = acc + plsc.load_gather(lut_v, [idx])
                rowsum = jnp.sum(acc)
                vec = jnp.where(
                    lax.iota(jnp.int32, LANES) == r, rowsum, vec
                )
            out_v[pl.ds(ci * CHUNK + g * LANES, LANES)] = vec
            return 0

        lax.fori_loop(0, CHUNK // LANES, group_body, 0)
        return 0

    lax.fori_loop(0, ROWS_PER_TILE // CHUNK, chunk_body, 0)
    pltpu.sync_copy(out_v, out_hbm.at[pl.ds(base_row, ROWS_PER_TILE)])


_sc_gather = functools.partial(
    pl.kernel,
    out_type=jax.ShapeDtypeStruct((BATCH,), jnp.float32),
    mesh=plsc.VectorSubcoreMesh(core_axis_name="c", subcore_axis_name="s"),
    compiler_params=pltpu.CompilerParams(needs_layout_passes=False),
    scratch_types=[
        pltpu.VMEM((TABLE,), jnp.float32),
        pltpu.VMEM((CHUNK * NUM_LUTS,), jnp.int32),
        pltpu.VMEM((ROWS_PER_TILE,), jnp.float32),
    ],
)(_sc_body)


def kernel(x, lut_weights):
    k = jnp.arange(NUM_INPUTS)
    p = ((k[:, None] // TUPLE_SIZE == jnp.arange(NUM_LUTS)[None, :])
         * (1 << (k % TUPLE_SIZE))[:, None]).astype(jnp.bfloat16)
    fidx = _pack_addresses(x, p).reshape(-1)
    return _sc_gather(fidx, lut_weights.reshape(-1))


# speedup vs baseline: 496.4438x; 1.2698x over previous
import functools

import jax
import jax.numpy as jnp
from jax import lax
from jax.experimental import pallas as pl
from jax.experimental.pallas import tpu as pltpu
from jax.experimental.pallas import tpu_sc as plsc

BATCH = 16384
NUM_INPUTS = 2048
TUPLE_SIZE = 8
NUM_LUTS = NUM_INPUTS // TUPLE_SIZE
LUT_ENTRIES = 1 << TUPLE_SIZE


TC_ROWS = 512


def _pack_body(x_ref, pt_ref, out_ref):
    xb = x_ref[...].astype(jnp.bfloat16)
    addr = lax.dot_general(
        pt_ref[...], xb,
        dimension_numbers=(((1,), (1,)), ((), ())),
        preferred_element_type=jnp.float32,
    )
    toff = lax.broadcasted_iota(jnp.int32, (NUM_LUTS, TC_ROWS), 0) * LUT_ENTRIES
    out_ref[...] = addr.astype(jnp.int32) + toff


def _pack_addresses(x, pt):
    return pl.pallas_call(
        _pack_body,
        grid=(BATCH // TC_ROWS,),
        in_specs=[
            pl.BlockSpec((TC_ROWS, NUM_INPUTS), lambda i: (i, 0)),
            pl.BlockSpec((NUM_LUTS, NUM_INPUTS), lambda i: (0, 0)),
        ],
        out_specs=pl.BlockSpec((NUM_LUTS, TC_ROWS), lambda i: (0, i)),
        out_shape=jax.ShapeDtypeStruct((NUM_LUTS, BATCH), jnp.int32),
    )(x, pt)


NW = 32
COLS_PER_TILE = BATCH // NW
CHUNK = 128
LANES = 16
GROUPS = CHUNK // LANES
TABLE = NUM_LUTS * LUT_ENTRIES


def _sc_body(fidx_hbm, lut_hbm, out_hbm, lut_v, idx_v, out_v):
    wid = lax.axis_index("s") * 2 + lax.axis_index("c")
    base_col = wid * COLS_PER_TILE
    pltpu.sync_copy(lut_hbm, lut_v)

    def chunk_body(ci, _):
        col0 = base_col + ci * CHUNK
        pltpu.sync_copy(fidx_hbm.at[:, pl.ds(col0, CHUNK)], idx_v)

        def t_body(t, accs):
            new = []
            for g in range(GROUPS):
                idx = idx_v[t, pl.ds(g * LANES, LANES)]
                new.append(accs[g] + plsc.load_gather(lut_v, [idx]))
            return tuple(new)

        zeros = jnp.zeros((LANES,), jnp.float32)
        accs = lax.fori_loop(0, NUM_LUTS, t_body, (zeros,) * GROUPS)
        for g in range(GROUPS):
            out_v[pl.ds(ci * CHUNK + g * LANES, LANES)] = accs[g]
        return 0

    lax.fori_loop(0, COLS_PER_TILE // CHUNK, chunk_body, 0)
    pltpu.sync_copy(out_v, out_hbm.at[pl.ds(base_col, COLS_PER_TILE)])


_sc_gather = functools.partial(
    pl.kernel,
    out_type=jax.ShapeDtypeStruct((BATCH,), jnp.float32),
    mesh=plsc.VectorSubcoreMesh(core_axis_name="c", subcore_axis_name="s"),
    compiler_params=pltpu.CompilerParams(needs_layout_passes=False),
    scratch_types=[
        pltpu.VMEM((TABLE,), jnp.float32),
        pltpu.VMEM((NUM_LUTS, CHUNK), jnp.int32),
        pltpu.VMEM((COLS_PER_TILE,), jnp.float32),
    ],
)(_sc_body)


def kernel(x, lut_weights):
    k = jnp.arange(NUM_INPUTS)
    pt = ((k[None, :] // TUPLE_SIZE == jnp.arange(NUM_LUTS)[:, None])
          * (1 << (k % TUPLE_SIZE))[None, :]).astype(jnp.bfloat16)
    fidx = _pack_addresses(x, pt)
    return _sc_gather(fidx, lut_weights.reshape(-1))
